# Initial kernel scaffold; baseline (speedup 1.0000x reference)
import functools

import jax
import jax.numpy as jnp
from jax import lax
from jax.experimental import pallas as pl
from jax.experimental.pallas import tpu as pltpu
from jax.experimental.pallas import tpu_sc as plsc

VOCAB = 1000000
EMBED_DIM = 32
NUM_CLASSES = 10
CPAD = 16

NC = 2
NS = 16
NW = NC * NS

ROW_BLK = 8000

IDX_GRP = 128
GRPS = 16
CHUNK = IDX_GRP * GRPS


def _project_body(t_ref, wt_ref, b_ref, o_ref):
    o_ref[...] = (
        jnp.dot(t_ref[...], wt_ref[...], preferred_element_type=jnp.float32)
        + b_ref[...]
    )


def _project_table(table, wt_pad, b_pad):
    grid = (VOCAB // ROW_BLK,)
    return pl.pallas_call(
        _project_body,
        grid=grid,
        in_specs=[
            pl.BlockSpec((ROW_BLK, EMBED_DIM), lambda i: (i, 0)),
            pl.BlockSpec((EMBED_DIM, CPAD), lambda i: (0, 0)),
            pl.BlockSpec((1, CPAD), lambda i: (0, 0)),
        ],
        out_specs=pl.BlockSpec((ROW_BLK, CPAD), lambda i: (i, 0)),
        out_shape=jax.ShapeDtypeStruct((VOCAB, CPAD), jnp.float32),
    )(table, wt_pad, b_pad)


def _gather_body(n_tok, ptable_hbm, idx_hbm, out_hbm, idx_v, rows_v, sem, gsem):
    wid = lax.axis_index("s") * NC + lax.axis_index("c")
    per_w = n_tok // NW
    base = wid * per_w
    n_chunks = per_w // CHUNK

    def step(i, _):
        off = base + i * CHUNK
        pltpu.sync_copy(idx_hbm.at[pl.ds(off // IDX_GRP, GRPS)], idx_v)
        for j in range(GRPS):
            pltpu.async_copy(
                ptable_hbm.at[idx_v.at[j]],
                rows_v.at[pl.ds(j * IDX_GRP, IDX_GRP)],
                gsem,
            )
        for j in range(GRPS):
            pltpu.make_async_copy(
                ptable_hbm.at[idx_v.at[j]],
                rows_v.at[pl.ds(j * IDX_GRP, IDX_GRP)],
                gsem,
            ).wait()
        pltpu.sync_copy(rows_v, out_hbm.at[pl.ds(off, CHUNK)])
        return 0

    lax.fori_loop(0, n_chunks, step, 0, unroll=False)


def _gather(ptable, idx_flat):
    n_tok = idx_flat.shape[0]
    idx2d = idx_flat.reshape(n_tok // IDX_GRP, IDX_GRP)
    mesh = plsc.VectorSubcoreMesh(core_axis_name="c", subcore_axis_name="s")
    kern = pl.kernel(
        functools.partial(_gather_body, n_tok),
        out_type=jax.ShapeDtypeStruct((n_tok, CPAD), jnp.float32),
        mesh=mesh,
        scratch_types=[
            pltpu.VMEM((GRPS, IDX_GRP), jnp.int32),
            pltpu.VMEM((CHUNK, CPAD), jnp.float32),
            pltpu.SemaphoreType.DMA,
            pltpu.SemaphoreType.DMA,
        ],
    )
    return kern(ptable, idx2d)


def kernel(x, table, W, b):
    wt_pad = jnp.zeros((EMBED_DIM, CPAD), jnp.float32).at[:, :NUM_CLASSES].set(W.T)
    b_pad = jnp.zeros((1, CPAD), jnp.float32).at[0, :NUM_CLASSES].set(b)
    ptable = _project_table(table, wt_pad, b_pad)
    bsz, seq = x.shape
    idx_flat = x.reshape(bsz * seq)
    rows = _gather(ptable, idx_flat)
    return rows[:, :NUM_CLASSES].reshape(bsz, seq, NUM_CLASSES)


# speedup vs baseline: 5.4731x; 5.4731x over previous
import functools

import jax
import jax.numpy as jnp
from jax import lax
from jax.experimental import pallas as pl
from jax.experimental.pallas import tpu as pltpu
from jax.experimental.pallas import tpu_sc as plsc

VOCAB = 1000000
EMBED_DIM = 32
NUM_CLASSES = 10
CPAD = 16

NC = 2
NS = 16
NW = NC * NS

IDX_GRP = 128
GRPS = 16
CHUNK = IDX_GRP * GRPS


def _project_body(t_ref, wt_ref, b_ref, o_ref):
    t = t_ref[...]
    wt = wt_ref[...]
    bb = b_ref[...]
    for j in range(8):
        pj = lax.dot_general(
            t[:, 0, j, :], wt,
            (((0,), (0,)), ((), ())),
            preferred_element_type=jnp.float32,
        ) + bb
        o_ref[0, j] = pj


def _project_table(table_t4, wt_pad, b_pad):
    return pl.pallas_call(
        _project_body,
        grid=(125,),
        in_specs=[
            pl.BlockSpec((32, 1, 8, 1000), lambda m: (0, m, 0, 0)),
            pl.BlockSpec((EMBED_DIM, CPAD), lambda m: (0, 0)),
            pl.BlockSpec((1, CPAD), lambda m: (0, 0)),
        ],
        out_specs=pl.BlockSpec((1, 8, 1000, CPAD), lambda m: (m, 0, 0, 0)),
        out_shape=jax.ShapeDtypeStruct((125, 8, 1000, CPAD), jnp.float32),
    )(table_t4, wt_pad, b_pad)


def _gather_body(n_tok, ptable_hbm, idx_hbm, out_hbm,
                 idx_a, idx_b, rows_a, rows_b, gsem_a, gsem_b):
    wid = lax.axis_index("s") * NC + lax.axis_index("c")
    per_w = n_tok // NW
    base_g = wid * (per_w // IDX_GRP)
    n_chunks = per_w // CHUNK

    def stage(idx_v, ci):
        g0 = pl.multiple_of(base_g + ci * GRPS, GRPS)
        pltpu.sync_copy(idx_hbm.at[pl.ds(g0, GRPS)], idx_v)

    def fire(idx_v, rows_v, sem):
        for j in range(GRPS):
            pltpu.async_copy(ptable_hbm.at[idx_v.at[j]], rows_v.at[j], sem)

    def drain(idx_v, rows_v, sem):
        for j in range(GRPS):
            pltpu.make_async_copy(ptable_hbm.at[idx_v.at[j]], rows_v.at[j], sem).wait()

    def write(rows_v, ci):
        g0 = pl.multiple_of(base_g + ci * GRPS, GRPS)
        pltpu.sync_copy(rows_v, out_hbm.at[pl.ds(g0, GRPS)])

    stage(idx_a, 0)
    fire(idx_a, rows_a, gsem_a)

    @pl.loop(0, n_chunks - 2, step=2)
    def _(i):
        stage(idx_b, i + 1)
        fire(idx_b, rows_b, gsem_b)
        drain(idx_a, rows_a, gsem_a)
        write(rows_a, i)
        stage(idx_a, i + 2)
        fire(idx_a, rows_a, gsem_a)
        drain(idx_b, rows_b, gsem_b)
        write(rows_b, i + 1)

    i_last = n_chunks - 2
    stage(idx_b, i_last + 1)
    fire(idx_b, rows_b, gsem_b)
    drain(idx_a, rows_a, gsem_a)
    write(rows_a, i_last)
    drain(idx_b, rows_b, gsem_b)
    write(rows_b, i_last + 1)


def _gather(ptable, idx2d):
    n_tok = idx2d.shape[0] * idx2d.shape[1]
    mesh = plsc.VectorSubcoreMesh(core_axis_name="c", subcore_axis_name="s")
    kern = pl.kernel(
        functools.partial(_gather_body, n_tok),
        out_type=jax.ShapeDtypeStruct((n_tok // IDX_GRP, IDX_GRP, CPAD), jnp.float32),
        mesh=mesh,
        scratch_types=[
            pltpu.VMEM((GRPS, IDX_GRP), jnp.int32),
            pltpu.VMEM((GRPS, IDX_GRP), jnp.int32),
            pltpu.VMEM((GRPS, IDX_GRP, CPAD), jnp.float32),
            pltpu.VMEM((GRPS, IDX_GRP, CPAD), jnp.float32),
            pltpu.SemaphoreType.DMA,
            pltpu.SemaphoreType.DMA,
        ],
        compiler_params=pltpu.CompilerParams(use_tc_tiling_on_sc=False),
    )
    return kern(ptable, idx2d)


def kernel(x, table, W, b):
    bsz, seq = x.shape
    n_tok = bsz * seq
    table_t4 = table.T.reshape(EMBED_DIM, 125, 8, VOCAB // 1000)
    wt_pad = jnp.zeros((EMBED_DIM, CPAD), jnp.float32).at[:, :NUM_CLASSES].set(W.T)
    b_pad = jnp.zeros((1, CPAD), jnp.float32).at[0, :NUM_CLASSES].set(b)
    ptable = _project_table(table_t4, wt_pad, b_pad).reshape(VOCAB, CPAD)
    idx2d = x.T.reshape(n_tok // IDX_GRP, IDX_GRP)
    rows = _gather(ptable, idx2d)
    rows = rows.reshape(seq, bsz, CPAD)
    return rows.transpose(1, 0, 2)[:, :, :NUM_CLASSES]


# speedup vs baseline: 5.5928x; 1.0219x over previous
import functools

import jax
import jax.numpy as jnp
from jax import lax
from jax.experimental import pallas as pl
from jax.experimental.pallas import tpu as pltpu
from jax.experimental.pallas import tpu_sc as plsc

VOCAB = 1000000
EMBED_DIM = 32
NUM_CLASSES = 10
CPAD = 16

NC = 2
NS = 16
NW = NC * NS

IDX_GRP = 128
GRPS = 20
CHUNK = IDX_GRP * GRPS


def _project_body(t_ref, wt_ref, b_ref, o_ref):
    t = t_ref[...]
    wt = wt_ref[...]
    bb = b_ref[...]
    for m in range(5):
        for j in range(8):
            pj = lax.dot_general(
                t[:, m, j, :], wt,
                (((0,), (0,)), ((), ())),
                preferred_element_type=jnp.float32,
            ) + bb
            o_ref[m, j] = pj


def _project_table(table_t4, wt_pad, b_pad):
    return pl.pallas_call(
        _project_body,
        grid=(25,),
        in_specs=[
            pl.BlockSpec((32, 5, 8, 1000), lambda m: (0, m, 0, 0)),
            pl.BlockSpec((EMBED_DIM, CPAD), lambda m: (0, 0)),
            pl.BlockSpec((1, CPAD), lambda m: (0, 0)),
        ],
        out_specs=pl.BlockSpec((5, 8, 1000, CPAD), lambda m: (m, 0, 0, 0)),
        out_shape=jax.ShapeDtypeStruct((125, 8, 1000, CPAD), jnp.float32),
    )(table_t4, wt_pad, b_pad)


def _gather_body(n_tok, ptable_hbm, idx_hbm, out_hbm,
                 idx_a, idx_b, rows_a, rows_b, gsem_a, gsem_b):
    wid = lax.axis_index("s") * NC + lax.axis_index("c")
    per_w = n_tok // NW
    base_g = wid * (per_w // IDX_GRP)
    n_chunks = per_w // CHUNK

    def stage(idx_v, ci):
        g0 = pl.multiple_of(base_g + ci * GRPS, GRPS)
        pltpu.sync_copy(idx_hbm.at[pl.ds(g0, GRPS)], idx_v)

    def fire(idx_v, rows_v, sem):
        for j in range(GRPS):
            pltpu.async_copy(ptable_hbm.at[idx_v.at[j]], rows_v.at[j], sem)

    def drain(idx_v, rows_v, sem):
        for j in range(GRPS):
            pltpu.make_async_copy(ptable_hbm.at[idx_v.at[j]], rows_v.at[j], sem).wait()

    def write(rows_v, ci):
        g0 = pl.multiple_of(base_g + ci * GRPS, GRPS)
        pltpu.sync_copy(rows_v, out_hbm.at[pl.ds(g0, GRPS)])

    stage(idx_a, 0)
    fire(idx_a, rows_a, gsem_a)

    @pl.loop(0, n_chunks - 2, step=2)
    def _(i):
        stage(idx_b, i + 1)
        fire(idx_b, rows_b, gsem_b)
        drain(idx_a, rows_a, gsem_a)
        write(rows_a, i)
        stage(idx_a, i + 2)
        fire(idx_a, rows_a, gsem_a)
        drain(idx_b, rows_b, gsem_b)
        write(rows_b, i + 1)

    i_last = n_chunks - 2
    stage(idx_b, i_last + 1)
    fire(idx_b, rows_b, gsem_b)
    drain(idx_a, rows_a, gsem_a)
    write(rows_a, i_last)
    drain(idx_b, rows_b, gsem_b)
    write(rows_b, i_last + 1)


def _gather(ptable, idx2d):
    n_tok = idx2d.shape[0] * idx2d.shape[1]
    mesh = plsc.VectorSubcoreMesh(core_axis_name="c", subcore_axis_name="s")
    kern = pl.kernel(
        functools.partial(_gather_body, n_tok),
        out_type=jax.ShapeDtypeStruct((n_tok // IDX_GRP, IDX_GRP, CPAD), jnp.float32),
        mesh=mesh,
        scratch_types=[
            pltpu.VMEM((GRPS, IDX_GRP), jnp.int32),
            pltpu.VMEM((GRPS, IDX_GRP), jnp.int32),
            pltpu.VMEM((GRPS, IDX_GRP, CPAD), jnp.float32),
            pltpu.VMEM((GRPS, IDX_GRP, CPAD), jnp.float32),
            pltpu.SemaphoreType.DMA,
            pltpu.SemaphoreType.DMA,
        ],
        compiler_params=pltpu.CompilerParams(use_tc_tiling_on_sc=False),
    )
    return kern(ptable, idx2d)


def kernel(x, table, W, b):
    bsz, seq = x.shape
    n_tok = bsz * seq
    table_t4 = table.T.reshape(EMBED_DIM, 125, 8, VOCAB // 1000)
    wt_pad = jnp.zeros((EMBED_DIM, CPAD), jnp.float32).at[:, :NUM_CLASSES].set(W.T)
    b_pad = jnp.zeros((1, CPAD), jnp.float32).at[0, :NUM_CLASSES].set(b)
    ptable = _project_table(table_t4, wt_pad, b_pad).reshape(VOCAB, CPAD)
    idx2d = x.T.reshape(n_tok // IDX_GRP, IDX_GRP)
    rows = _gather(ptable, idx2d)
    rows = rows.reshape(seq, bsz, CPAD)
    return rows.transpose(1, 0, 2)[:, :, :NUM_CLASSES]
